# baseline (device time: 405391 ns/iter reference)
import jax
import jax.numpy as jnp
from jax import lax
from jax.experimental import pallas as pl
from jax.experimental.pallas import tpu as pltpu

N_DEV = 4
HQ = 8
DH = 128
SQ = 2048
SKV_SH = 2048
D_MODEL = 1024
SCALE = 0.08838834764831843
LOG2E = 1.4426950408889634


def _all_peer_barrier(my):
    barrier = pltpu.get_barrier_semaphore()
    for d in range(1, N_DEV):
        peer = lax.rem(my + d, N_DEV)
        pl.semaphore_signal(
            barrier, inc=1, device_id=(peer,),
            device_id_type=pl.DeviceIdType.MESH,
        )
    pl.semaphore_wait(barrier, N_DEV - 1)


def _a2a_body(k_ref, v_ref, ko_ref, vo_ref, copy_sems, send_sems, recv_sems):
    my = lax.axis_index("i")
    _all_peer_barrier(my)

    copies = []
    for t, (src, dst) in enumerate(((k_ref, ko_ref), (v_ref, vo_ref))):
        c = pltpu.make_async_copy(
            src.at[:, :, :, pl.ds(my * HQ, HQ), :], dst.at[my],
            copy_sems.at[t],
        )
        c.start()
        copies.append(c)

    sends, recvs = [], []
    for d in range(1, N_DEV):
        peer = lax.rem(my + d, N_DEV)
        src_dev = lax.rem(my - d + N_DEV, N_DEV)
        for t, (src, dst) in enumerate(((k_ref, ko_ref), (v_ref, vo_ref))):
            i = t * (N_DEV - 1) + (d - 1)
            send = pltpu.make_async_remote_copy(
                src_ref=src.at[:, :, :, pl.ds(peer * HQ, HQ), :],
                dst_ref=dst.at[my],
                send_sem=send_sems.at[i],
                recv_sem=recv_sems.at[i],
                device_id=(peer,),
                device_id_type=pl.DeviceIdType.MESH,
            )
            send.start()
            sends.append(send)
            recv = pltpu.make_async_remote_copy(
                src_ref=src.at[:, :, :, pl.ds(0, HQ), :],
                dst_ref=dst.at[src_dev],
                send_sem=send_sems.at[i],
                recv_sem=recv_sems.at[i],
                device_id=(peer,),
                device_id_type=pl.DeviceIdType.MESH,
            )
            recvs.append(recv)

    for c in copies:
        c.wait()
    for s in sends:
        s.wait_send()
    for r in recvs:
        r.wait_recv()


def _a2a_kv(k, v):
    return pl.pallas_call(
        _a2a_body,
        out_shape=(
            jax.ShapeDtypeStruct((N_DEV, 8, 4, 64, HQ, DH), k.dtype),
            jax.ShapeDtypeStruct((N_DEV, 8, 4, 64, HQ, DH), v.dtype),
        ),
        in_specs=[pl.BlockSpec(memory_space=pl.ANY)] * 2,
        out_specs=(
            pl.BlockSpec(memory_space=pl.ANY),
            pl.BlockSpec(memory_space=pl.ANY),
        ),
        scratch_shapes=[
            pltpu.SemaphoreType.DMA((2,)),
            pltpu.SemaphoreType.DMA((2 * (N_DEV - 1),)),
            pltpu.SemaphoreType.DMA((2 * (N_DEV - 1),)),
        ],
        compiler_params=pltpu.CompilerParams(collective_id=0),
    )(k, v)


def _ar_body(p_ref, o_ref, s16, q16, rs_recv, ag_recv,
             rs_ssem, rs_rsem, ag_ssem, ag_rsem):
    my = lax.axis_index("i")
    _all_peer_barrier(my)

    s16[...] = p_ref[...].astype(jnp.bfloat16)

    rs_sends = []
    for d in range(1, N_DEV):
        peer = lax.rem(my + d, N_DEV)
        send = pltpu.make_async_remote_copy(
            src_ref=s16.at[peer],
            dst_ref=rs_recv.at[d - 1],
            send_sem=rs_ssem.at[d - 1],
            recv_sem=rs_rsem.at[d - 1],
            device_id=(peer,),
            device_id_type=pl.DeviceIdType.MESH,
        )
        send.start()
        rs_sends.append(send)
    for d in range(1, N_DEV):
        recv = pltpu.make_async_remote_copy(
            src_ref=s16.at[0],
            dst_ref=rs_recv.at[d - 1],
            send_sem=rs_ssem.at[d - 1],
            recv_sem=rs_rsem.at[d - 1],
            device_id=(my,),
            device_id_type=pl.DeviceIdType.MESH,
        )
        recv.wait_recv()

    q = p_ref[my]
    for j in range(N_DEV - 1):
        q = q + rs_recv[j].astype(jnp.float32)
    o_ref[my] = q
    q16[...] = q.astype(jnp.bfloat16)

    ag_sends = []
    for d in range(1, N_DEV):
        peer = lax.rem(my + d, N_DEV)
        send = pltpu.make_async_remote_copy(
            src_ref=q16,
            dst_ref=ag_recv.at[d - 1],
            send_sem=ag_ssem.at[d - 1],
            recv_sem=ag_rsem.at[d - 1],
            device_id=(peer,),
            device_id_type=pl.DeviceIdType.MESH,
        )
        send.start()
        ag_sends.append(send)
    for d in range(1, N_DEV):
        src_dev = lax.rem(my - d + N_DEV, N_DEV)
        recv = pltpu.make_async_remote_copy(
            src_ref=q16,
            dst_ref=ag_recv.at[d - 1],
            send_sem=ag_ssem.at[d - 1],
            recv_sem=ag_rsem.at[d - 1],
            device_id=(my,),
            device_id_type=pl.DeviceIdType.MESH,
        )
        recv.wait_recv()
        o_ref[src_dev] = ag_recv[d - 1].astype(jnp.float32)

    for s in rs_sends:
        s.wait_send()
    for s in ag_sends:
        s.wait_send()


def _allreduce(partial):
    qd = SQ // N_DEV
    return pl.pallas_call(
        _ar_body,
        out_shape=jax.ShapeDtypeStruct((N_DEV, qd, D_MODEL), jnp.float32),
        in_specs=[pl.BlockSpec(memory_space=pltpu.VMEM)],
        out_specs=pl.BlockSpec(memory_space=pltpu.VMEM),
        scratch_shapes=[
            pltpu.VMEM((N_DEV, qd, D_MODEL), jnp.bfloat16),
            pltpu.VMEM((qd, D_MODEL), jnp.bfloat16),
            pltpu.VMEM((N_DEV - 1, qd, D_MODEL), jnp.bfloat16),
            pltpu.VMEM((N_DEV - 1, qd, D_MODEL), jnp.bfloat16),
            pltpu.SemaphoreType.DMA((N_DEV - 1,)),
            pltpu.SemaphoreType.DMA((N_DEV - 1,)),
            pltpu.SemaphoreType.DMA((N_DEV - 1,)),
            pltpu.SemaphoreType.DMA((N_DEV - 1,)),
        ],
        compiler_params=pltpu.CompilerParams(collective_id=1),
    )(partial)


def _attn_body(x_ref, wq_ref, k_ref, v_ref, wo_ref, o_ref):
    xb = x_ref[:, 0].reshape(512, 1024)
    total = jnp.zeros((512, 1024), jnp.float32)
    for h in range(HQ):
        q = jnp.dot(
            xb, wq_ref[:, h * DH:(h + 1) * DH],
            preferred_element_type=jnp.float32,
        )
        k = k_ref[:, :, 0, :, h, :].reshape(2048, DH)
        v = v_ref[:, :, 0, :, h, :].reshape(2048, DH)
        qs = (q * (SCALE * LOG2E)).astype(jnp.bfloat16)
        s = jnp.dot(qs, k.T, preferred_element_type=jnp.float32)
        w = jnp.exp2(s)
        l = jnp.sum(w, axis=-1, keepdims=True)
        ctx = jnp.dot(
            w.astype(jnp.bfloat16), v, preferred_element_type=jnp.float32
        ) / l
        total += jnp.dot(
            ctx.astype(jnp.bfloat16), wo_ref[h * DH:(h + 1) * DH, :],
            preferred_element_type=jnp.float32,
        )
    o_ref[...] = total.reshape(8, 1, 64, 1024)


def _attention(x4, Wq, kf, vf, Wo):
    kv_spec = pl.BlockSpec(
        (N_DEV, 8, 1, 64, HQ, DH), lambda r: (0, 0, r, 0, 0, 0)
    )
    return pl.pallas_call(
        _attn_body,
        grid=(4,),
        in_specs=[
            pl.BlockSpec((8, 1, 64, 1024), lambda r: (0, r, 0, 0)),
            pl.BlockSpec((1024, 1024), lambda r: (0, 0)),
            kv_spec,
            kv_spec,
            pl.BlockSpec((1024, 1024), lambda r: (0, 0)),
        ],
        out_specs=pl.BlockSpec((8, 1, 64, 1024), lambda r: (0, r, 0, 0)),
        out_shape=jax.ShapeDtypeStruct((8, 4, 64, 1024), jnp.float32),
        compiler_params=pltpu.CompilerParams(
            vmem_limit_bytes=56 * 1024 * 1024
        ),
    )(x4, Wq, kf, vf, Wo)


def kernel(x, Wq, K_ext, V_ext, Wo):
    k16 = K_ext[0].astype(jnp.bfloat16).reshape(8, 4, 64, 32, DH)
    v16 = V_ext[0].astype(jnp.bfloat16).reshape(8, 4, 64, 32, DH)
    kf, vf = _a2a_kv(k16, v16)

    x4 = x[0].astype(jnp.bfloat16).reshape(8, 4, 64, D_MODEL)
    wq16 = Wq.astype(jnp.bfloat16)
    wo16 = Wo.astype(jnp.bfloat16)
    partial = _attention(x4, wq16, kf, vf, wo16)
    full = _allreduce(partial.reshape(N_DEV, SQ // N_DEV, D_MODEL))
    return full.reshape(1, SQ, D_MODEL)


# device time: 380167 ns/iter; 1.0663x vs baseline; 1.0663x over previous
import jax
import jax.numpy as jnp
from jax import lax
from jax.experimental import pallas as pl
from jax.experimental.pallas import tpu as pltpu

N_DEV = 4
HQ = 8
DH = 128
SQ = 2048
SKV_SH = 2048
D_MODEL = 1024
SCALE = 0.08838834764831843
LOG2E = 1.4426950408889634


def _all_peer_barrier(my):
    barrier = pltpu.get_barrier_semaphore()
    for d in range(1, N_DEV):
        peer = lax.rem(my + d, N_DEV)
        pl.semaphore_signal(
            barrier, inc=1, device_id=(peer,),
            device_id_type=pl.DeviceIdType.MESH,
        )
    pl.semaphore_wait(barrier, N_DEV - 1)


def _a2a_body(k_ref, v_ref, ko_ref, vo_ref, copy_sems, send_sems, recv_sems):
    my = lax.axis_index("i")
    _all_peer_barrier(my)

    copies = []
    for t, (src, dst) in enumerate(((k_ref, ko_ref), (v_ref, vo_ref))):
        c = pltpu.make_async_copy(
            src.at[pl.ds(my * HQ, HQ)], dst.at[my], copy_sems.at[t]
        )
        c.start()
        copies.append(c)

    sends, recvs = [], []
    for d in range(1, N_DEV):
        peer = lax.rem(my + d, N_DEV)
        src_dev = lax.rem(my - d + N_DEV, N_DEV)
        for t, (src, dst) in enumerate(((k_ref, ko_ref), (v_ref, vo_ref))):
            i = t * (N_DEV - 1) + (d - 1)
            send = pltpu.make_async_remote_copy(
                src_ref=src.at[pl.ds(peer * HQ, HQ)],
                dst_ref=dst.at[my],
                send_sem=send_sems.at[i],
                recv_sem=recv_sems.at[i],
                device_id=(peer,),
                device_id_type=pl.DeviceIdType.MESH,
            )
            send.start()
            sends.append(send)
            recv = pltpu.make_async_remote_copy(
                src_ref=src.at[pl.ds(0, HQ)],
                dst_ref=dst.at[src_dev],
                send_sem=send_sems.at[i],
                recv_sem=recv_sems.at[i],
                device_id=(peer,),
                device_id_type=pl.DeviceIdType.MESH,
            )
            recvs.append(recv)

    for c in copies:
        c.wait()
    for s in sends:
        s.wait_send()
    for r in recvs:
        r.wait_recv()


def _a2a_kv(k, v):
    return pl.pallas_call(
        _a2a_body,
        out_shape=(
            jax.ShapeDtypeStruct((N_DEV, HQ, SKV_SH, DH), k.dtype),
            jax.ShapeDtypeStruct((N_DEV, HQ, SKV_SH, DH), v.dtype),
        ),
        in_specs=[pl.BlockSpec(memory_space=pl.ANY)] * 2,
        out_specs=(
            pl.BlockSpec(memory_space=pl.ANY),
            pl.BlockSpec(memory_space=pl.ANY),
        ),
        scratch_shapes=[
            pltpu.SemaphoreType.DMA((2,)),
            pltpu.SemaphoreType.DMA((2 * (N_DEV - 1),)),
            pltpu.SemaphoreType.DMA((2 * (N_DEV - 1),)),
        ],
        compiler_params=pltpu.CompilerParams(collective_id=0),
    )(k, v)


def _ar_body(p_ref, o_ref, s16, q16, rs_recv, ag_recv,
             rs_ssem, rs_rsem, ag_ssem, ag_rsem):
    my = lax.axis_index("i")
    _all_peer_barrier(my)

    s16[...] = p_ref[...].astype(jnp.bfloat16)

    rs_sends = []
    for d in range(1, N_DEV):
        peer = lax.rem(my + d, N_DEV)
        send = pltpu.make_async_remote_copy(
            src_ref=s16.at[peer],
            dst_ref=rs_recv.at[d - 1],
            send_sem=rs_ssem.at[d - 1],
            recv_sem=rs_rsem.at[d - 1],
            device_id=(peer,),
            device_id_type=pl.DeviceIdType.MESH,
        )
        send.start()
        rs_sends.append(send)
    for d in range(1, N_DEV):
        recv = pltpu.make_async_remote_copy(
            src_ref=s16.at[0],
            dst_ref=rs_recv.at[d - 1],
            send_sem=rs_ssem.at[d - 1],
            recv_sem=rs_rsem.at[d - 1],
            device_id=(my,),
            device_id_type=pl.DeviceIdType.MESH,
        )
        recv.wait_recv()

    q = p_ref[my]
    for j in range(N_DEV - 1):
        q = q + rs_recv[j].astype(jnp.float32)
    o_ref[my] = q
    q16[...] = q.astype(jnp.bfloat16)

    ag_sends = []
    for d in range(1, N_DEV):
        peer = lax.rem(my + d, N_DEV)
        send = pltpu.make_async_remote_copy(
            src_ref=q16,
            dst_ref=ag_recv.at[d - 1],
            send_sem=ag_ssem.at[d - 1],
            recv_sem=ag_rsem.at[d - 1],
            device_id=(peer,),
            device_id_type=pl.DeviceIdType.MESH,
        )
        send.start()
        ag_sends.append(send)
    for d in range(1, N_DEV):
        src_dev = lax.rem(my - d + N_DEV, N_DEV)
        recv = pltpu.make_async_remote_copy(
            src_ref=q16,
            dst_ref=ag_recv.at[d - 1],
            send_sem=ag_ssem.at[d - 1],
            recv_sem=ag_rsem.at[d - 1],
            device_id=(my,),
            device_id_type=pl.DeviceIdType.MESH,
        )
        recv.wait_recv()
        o_ref[src_dev] = ag_recv[d - 1].astype(jnp.float32)

    for s in rs_sends:
        s.wait_send()
    for s in ag_sends:
        s.wait_send()


def _allreduce(partial):
    qd = SQ // N_DEV
    return pl.pallas_call(
        _ar_body,
        out_shape=jax.ShapeDtypeStruct((N_DEV, qd, D_MODEL), jnp.float32),
        in_specs=[pl.BlockSpec(memory_space=pltpu.VMEM)],
        out_specs=pl.BlockSpec(memory_space=pltpu.VMEM),
        scratch_shapes=[
            pltpu.VMEM((N_DEV, qd, D_MODEL), jnp.bfloat16),
            pltpu.VMEM((qd, D_MODEL), jnp.bfloat16),
            pltpu.VMEM((N_DEV - 1, qd, D_MODEL), jnp.bfloat16),
            pltpu.VMEM((N_DEV - 1, qd, D_MODEL), jnp.bfloat16),
            pltpu.SemaphoreType.DMA((N_DEV - 1,)),
            pltpu.SemaphoreType.DMA((N_DEV - 1,)),
            pltpu.SemaphoreType.DMA((N_DEV - 1,)),
            pltpu.SemaphoreType.DMA((N_DEV - 1,)),
        ],
        compiler_params=pltpu.CompilerParams(collective_id=1),
    )(partial)


def _attn_body(x_ref, wq_ref, k_ref, v_ref, wo_ref, o_ref):
    h = pl.program_id(1)
    xb = x_ref[:, 0].reshape(512, 1024)
    q = jnp.dot(xb, wq_ref[...], preferred_element_type=jnp.float32)
    k = k_ref[:, 0, :, 0, :, :].reshape(2048, DH)
    v = v_ref[:, 0, :, 0, :, :].reshape(2048, DH)
    qs = (q * (SCALE * LOG2E)).astype(jnp.bfloat16)
    s = jnp.dot(qs, k.T, preferred_element_type=jnp.float32)
    w = jnp.exp2(s)
    l = jnp.sum(w, axis=-1, keepdims=True)
    ctx = jnp.dot(
        w.astype(jnp.bfloat16), v, preferred_element_type=jnp.float32
    ) / l
    contrib = jnp.dot(
        ctx.astype(jnp.bfloat16), wo_ref[...],
        preferred_element_type=jnp.float32,
    )
    contrib = contrib.reshape(8, 1, 64, 1024)

    @pl.when(h == 0)
    def _():
        o_ref[...] = contrib

    @pl.when(h != 0)
    def _():
        o_ref[...] += contrib


def _attention(x4, Wq, kf6, vf6, Wo):
    kv_spec = pl.BlockSpec(
        (N_DEV, 1, 8, 1, 64, DH), lambda r, h: (0, h, 0, r, 0, 0)
    )
    return pl.pallas_call(
        _attn_body,
        grid=(4, HQ),
        in_specs=[
            pl.BlockSpec((8, 1, 64, 1024), lambda r, h: (0, r, 0, 0)),
            pl.BlockSpec((1024, DH), lambda r, h: (0, h)),
            kv_spec,
            kv_spec,
            pl.BlockSpec((DH, 1024), lambda r, h: (h, 0)),
        ],
        out_specs=pl.BlockSpec((8, 1, 64, 1024), lambda r, h: (0, r, 0, 0)),
        out_shape=jax.ShapeDtypeStruct((8, 4, 64, 1024), jnp.float32),
    )(x4, Wq, kf6, vf6, Wo)


def kernel(x, Wq, K_ext, V_ext, Wo):
    kT = K_ext[0].astype(jnp.bfloat16).transpose(1, 0, 2)
    vT = V_ext[0].astype(jnp.bfloat16).transpose(1, 0, 2)
    kf, vf = _a2a_kv(kT, vT)

    x4 = x[0].astype(jnp.bfloat16).reshape(8, 4, 64, D_MODEL)
    kf6 = kf.reshape(N_DEV, HQ, 8, 4, 64, DH)
    vf6 = vf.reshape(N_DEV, HQ, 8, 4, 64, DH)

    wq16 = Wq.astype(jnp.bfloat16)
    wo16 = Wo.astype(jnp.bfloat16)
    partial = _attention(x4, wq16, kf6, vf6, wo16)
    full = _allreduce(partial.reshape(N_DEV, SQ // N_DEV, D_MODEL))
    return full.reshape(1, SQ, D_MODEL)
